# baseline (speedup 1.0000x reference)
import jax
import jax.numpy as jnp
from jax.experimental import pallas as pl
from jax.experimental.pallas import tpu as pltpu

_F32 = jnp.float32
_ANY = pl.ANY
_VMEM = pltpu.MemorySpace.VMEM


def _make_body(B, N, P, D, DOUT):
    BLK = N
    PB = P * N
    M = B * PB
    C = 3 * DOUT

    def body(x1_hbm, x2_hbm, wx_ref, wn_ref, wr_ref,
             bx_ref, bn_ref, br_ref, g_ref, bt_ref,
             o1_hbm, o2_hbm, xs1, xs2, hs1, hs2, sem):
        in_copies = []
        for p in range(P):
            for src, dst in ((x1_hbm, xs1), (x2_hbm, xs2)):
                cp = pltpu.make_async_copy(
                    src.at[:, :, p, :], dst.at[:, p, :, :], sem)
                cp.start()
                in_copies.append(cp)
        for cp in in_copies:
            cp.wait()

        att1 = []
        att2t = []
        for b in range(B):
            num = jnp.zeros((N, N), _F32)
            sq1 = jnp.zeros((N, 1), _F32)
            sq2 = jnp.zeros((N, 1), _F32)
            for p in range(P):
                a1 = xs1[b, p]
                a2 = xs2[b, p]
                num += jax.lax.dot_general(
                    a1, a2, (((1,), (1,)), ((), ())),
                    preferred_element_type=_F32)
                sq1 += jnp.sum(a1 * a1, axis=1, keepdims=True)
                sq2 += jnp.sum(a2 * a2, axis=1, keepdims=True)
            n1 = jnp.maximum(jnp.sqrt(sq1), 1e-6)
            n2 = jnp.maximum(jnp.sqrt(sq2), 1e-6)
            sim = num / (n1 * n2.T)
            m1 = jnp.max(sim, axis=1, keepdims=True)
            e1 = jnp.exp(sim - m1)
            att1.append(e1 / jnp.sum(e1, axis=1, keepdims=True))
            m2 = jnp.max(sim, axis=0, keepdims=True)
            e2 = jnp.exp(sim - m2)
            att2t.append(e2 / jnp.sum(e2, axis=0, keepdims=True))

        wx = wx_ref[:]
        wn = wn_ref[:]
        wr = wr_ref[:]
        bx = bx_ref[:]
        bn = bn_ref[:]
        br = br_ref[:]
        g = g_ref[:]
        bt = bt_ref[:]
        inv = 1.0 / (N - 1)

        X1 = xs1[:].reshape(M, D)
        X2 = xs2[:].reshape(M, D)
        out_copies = []
        for X, Xo, att, tr, oref, ohbm in (
                (X1, X2, att1, False, hs1, o1_hbm),
                (X2, X1, att2t, True, hs2, o2_hbm)):
            S = jnp.dot(X, wx, preferred_element_type=_F32) + bx
            rn2 = jnp.sum(S * S, axis=1, keepdims=True)
            oref[:, :, :, 0:DOUT] = S.reshape(B, P, N, DOUT)

            Z = jnp.dot(X, wn, preferred_element_type=_F32)
            Z3 = Z.reshape(B * P, BLK, DOUT)
            s = jnp.sum(Z3, axis=1, keepdims=True)
            XN = ((s - Z3) * inv).reshape(M, DOUT) + bn
            rn2 += jnp.sum(XN * XN, axis=1, keepdims=True)
            oref[:, :, :, DOUT:2 * DOUT] = XN.reshape(B, P, N, DOUT)

            Zs = jnp.dot(X, wr, preferred_element_type=_F32)
            Zo = jnp.dot(Xo, wr, preferred_element_type=_F32)
            mu_parts = []
            for b in range(B):
                a = att[b]
                for p in range(P):
                    r = b * PB + p * BLK
                    if not tr:
                        c = jnp.dot(a, Zo[r:r + BLK, :],
                                    preferred_element_type=_F32)
                    else:
                        c = jax.lax.dot_general(
                            a, Zo[r:r + BLK, :], (((0,), (0,)), ((), ())),
                            preferred_element_type=_F32)
                    mu_parts.append(Zs[r:r + BLK, :] - c + br)
            MUS = jnp.concatenate(mu_parts, axis=0)
            rn2 += jnp.sum(MUS * MUS, axis=1, keepdims=True)
            oref[:, :, :, 2 * DOUT:3 * DOUT] = MUS.reshape(B, P, N, DOUT)

            rinv = (1.0 / jnp.maximum(jnp.sqrt(rn2), 1e-12)).reshape(B, P, N, 1)
            h = jnp.maximum(oref[:] * rinv, 0.0)
            oref[:] = h
            h2 = h.reshape(M, C)
            sm = jnp.sum(h2, axis=0, keepdims=True)
            sq = jnp.sum(h2 * h2, axis=0, keepdims=True)
            mean = sm * (1.0 / M)
            var = jnp.maximum(sq * (1.0 / M) - mean * mean, 0.0)
            scale = (g * jax.lax.rsqrt(var + 1e-5)).reshape(1, 1, 1, C)
            shift = (bt - mean * g * jax.lax.rsqrt(var + 1e-5)).reshape(1, 1, 1, C)
            oref[:] = oref[:] * scale + shift

            for p in range(P):
                cp = pltpu.make_async_copy(
                    oref.at[:, p, :, :], ohbm.at[:, :, p, :], sem)
                cp.start()
                out_copies.append(cp)
        for cp in out_copies:
            cp.wait()

    return body


def kernel(x1, x2, adj1, adj2, Wx_w, Wx_b, Wn_w, Wn_b, Wr_w, Wr_b, gamma, beta):
    B, N, P, D = x1.shape
    DOUT = Wx_w.shape[0]
    C = 3 * DOUT

    vspec = pl.BlockSpec(memory_space=_VMEM)
    aspec = pl.BlockSpec(memory_space=_ANY)

    out1, out2 = pl.pallas_call(
        _make_body(B, N, P, D, DOUT),
        in_specs=[aspec, aspec] + [vspec] * 8,
        out_specs=(aspec, aspec),
        out_shape=(
            jax.ShapeDtypeStruct((B, N, P, C), jnp.float32),
            jax.ShapeDtypeStruct((B, N, P, C), jnp.float32),
        ),
        scratch_shapes=[
            pltpu.VMEM((B, P, N, D), jnp.float32),
            pltpu.VMEM((B, P, N, D), jnp.float32),
            pltpu.VMEM((B, P, N, C), jnp.float32),
            pltpu.VMEM((B, P, N, C), jnp.float32),
            pltpu.SemaphoreType.DMA,
        ],
    )(x1, x2,
      Wx_w.T, Wn_w.T, Wr_w.T,
      Wx_b.reshape(1, DOUT), Wn_b.reshape(1, DOUT), Wr_b.reshape(1, DOUT),
      gamma.reshape(1, C), beta.reshape(1, C))

    return (out1, out2)
